# baseline (device time: 33140 ns/iter reference)
import jax
import jax.numpy as jnp
from jax import lax
from jax.experimental import pallas as pl
from jax.experimental.pallas import tpu as pltpu

N_DEV = 4
B = 2
SQ_PER = 128
SKV_PER = 128
HQ = 4
DH = 64
HD = HQ * DH
BLK = 64


def kernel(x, Wq, K_ext, V_ext, Wo):
    def body(x_ref, wq_ref, k_ref, v_ref, wo_ref, out_ref,
             kbuf, vbuf, ksend, krecv, vsend, vrecv):
        my = lax.axis_index("i")
        left = lax.rem(my + N_DEV - 1, N_DEV)
        right = lax.rem(my + 1, N_DEV)

        barrier_sem = pltpu.get_barrier_semaphore()
        for nbr in (left, right):
            pl.semaphore_signal(
                barrier_sem, inc=1,
                device_id=(nbr,), device_id_type=pl.DeviceIdType.MESH,
            )
        pl.semaphore_wait(barrier_sem, 2)

        kbuf[my] = k_ref[...].reshape(B, SKV_PER, HD)
        vbuf[my] = v_ref[...].reshape(B, SKV_PER, HD)

        for h in range(N_DEV - 1):
            chunk = lax.rem(my - h + N_DEV, N_DEV)
            rk = pltpu.make_async_remote_copy(
                src_ref=kbuf.at[chunk], dst_ref=kbuf.at[chunk],
                send_sem=ksend.at[h], recv_sem=krecv.at[h],
                device_id=(right,), device_id_type=pl.DeviceIdType.MESH,
            )
            rv = pltpu.make_async_remote_copy(
                src_ref=vbuf.at[chunk], dst_ref=vbuf.at[chunk],
                send_sem=vsend.at[h], recv_sem=vrecv.at[h],
                device_id=(right,), device_id_type=pl.DeviceIdType.MESH,
            )
            rk.start()
            rv.start()
            rk.wait()
            rv.wait()

        SKV = N_DEV * SKV_PER
        row = lax.broadcasted_iota(jnp.int32, (SQ_PER, SKV), 0)
        col = lax.broadcasted_iota(jnp.int32, (SQ_PER, SKV), 1)
        qblk = row // BLK + 2 * my
        kblk = col // BLK
        mask = (qblk == kblk) | (kblk == 0) | (lax.rem(qblk + kblk, 3) == 0)
        neg = jnp.float32(-1e9)

        for b in range(B):
            q_b = jnp.dot(x_ref[b], wq_ref[...],
                          preferred_element_type=jnp.float32)
            ctx_parts = []
            for hh in range(HQ):
                q_h = q_b[:, hh * DH:(hh + 1) * DH]
                s_parts = []
                for c in range(N_DEV):
                    k_c = kbuf[c, b][:, hh * DH:(hh + 1) * DH]
                    s_parts.append(lax.dot_general(
                        q_h, k_c, (((1,), (1,)), ((), ())),
                        preferred_element_type=jnp.float32))
                scores = jnp.concatenate(s_parts, axis=1) * 0.125
                scores = jnp.where(mask, scores, neg)
                m = jnp.max(scores, axis=1, keepdims=True)
                w = jnp.exp(scores - m)
                w = w / jnp.sum(w, axis=1, keepdims=True)
                ctx_h = jnp.zeros((SQ_PER, DH), jnp.float32)
                for c in range(N_DEV):
                    v_c = vbuf[c, b][:, hh * DH:(hh + 1) * DH]
                    ctx_h = ctx_h + jnp.dot(
                        w[:, c * SKV_PER:(c + 1) * SKV_PER], v_c,
                        preferred_element_type=jnp.float32)
                ctx_parts.append(ctx_h)
            ctx = jnp.concatenate(ctx_parts, axis=1)
            out_ref[b] = jnp.dot(ctx, wo_ref[...],
                                 preferred_element_type=jnp.float32)

    return pl.pallas_call(
        body,
        out_shape=jax.ShapeDtypeStruct((B, SQ_PER, N_DEV * SKV_PER), jnp.float32),
        in_specs=[pl.BlockSpec(memory_space=pltpu.VMEM)] * 5,
        out_specs=pl.BlockSpec(memory_space=pltpu.VMEM),
        scratch_shapes=[
            pltpu.VMEM((N_DEV, B, SKV_PER, HD), jnp.float32),
            pltpu.VMEM((N_DEV, B, SKV_PER, HD), jnp.float32),
            pltpu.SemaphoreType.DMA((N_DEV - 1,)),
            pltpu.SemaphoreType.DMA((N_DEV - 1,)),
            pltpu.SemaphoreType.DMA((N_DEV - 1,)),
            pltpu.SemaphoreType.DMA((N_DEV - 1,)),
        ],
        compiler_params=pltpu.CompilerParams(collective_id=0),
    )(x, Wq, K_ext, V_ext, Wo)


# device time: 24764 ns/iter; 1.3382x vs baseline; 1.3382x over previous
import jax
import jax.numpy as jnp
from jax import lax
from jax.experimental import pallas as pl
from jax.experimental.pallas import tpu as pltpu

N_DEV = 4
B = 2
SQ_PER = 128
SKV_PER = 128
HQ = 4
DH = 64
HD = HQ * DH
BLK = 64


def kernel(x, Wq, K_ext, V_ext, Wo):
    def body(x_ref, wq_ref, k_ref, v_ref, wo_ref, out_ref,
             kbuf, vbuf, sr, rr, sl, rl):
        my = lax.axis_index("i")
        left = lax.rem(my + N_DEV - 1, N_DEV)
        right = lax.rem(my + 1, N_DEV)

        barrier_sem = pltpu.get_barrier_semaphore()
        for nbr in (left, right):
            pl.semaphore_signal(
                barrier_sem, inc=1,
                device_id=(nbr,), device_id_type=pl.DeviceIdType.MESH,
            )
        pl.semaphore_wait(barrier_sem, 2)

        kbuf[my] = k_ref[...].reshape(B, SKV_PER, HD)
        vbuf[my] = v_ref[...].reshape(B, SKV_PER, HD)

        def start_hop(h):
            cr = lax.rem(my - h + N_DEV, N_DEV)
            cl = lax.rem(my + h, N_DEV)
            rdmas = []
            for buf, flow, sem_i in ((kbuf, 0, 0), (vbuf, 1, 1)):
                rdmas.append(pltpu.make_async_remote_copy(
                    src_ref=buf.at[cr, 0], dst_ref=buf.at[cr, 0],
                    send_sem=sr.at[h, sem_i], recv_sem=rr.at[h, sem_i],
                    device_id=(right,), device_id_type=pl.DeviceIdType.MESH,
                ))
                rdmas.append(pltpu.make_async_remote_copy(
                    src_ref=buf.at[cl, 1], dst_ref=buf.at[cl, 1],
                    send_sem=sl.at[h, sem_i], recv_sem=rl.at[h, sem_i],
                    device_id=(left,), device_id_type=pl.DeviceIdType.MESH,
                ))
            for r in rdmas:
                r.start()
            return rdmas

        rdmas = start_hop(0)

        q_all = [jnp.dot(x_ref[b], wq_ref[...],
                         preferred_element_type=jnp.float32)
                 for b in range(B)]

        SKV = N_DEV * SKV_PER
        row = lax.broadcasted_iota(jnp.int32, (SQ_PER, SKV), 0)
        col = lax.broadcasted_iota(jnp.int32, (SQ_PER, SKV), 1)
        qblk = row // BLK + 2 * my
        kblk = col // BLK
        mask = (qblk == kblk) | (kblk == 0) | (lax.rem(qblk + kblk, 3) == 0)
        neg = jnp.float32(-1e9)

        for r in rdmas:
            r.wait()
        for h in range(1, N_DEV - 1):
            rdmas = start_hop(h)
            for r in rdmas:
                r.wait()

        for b in range(B):
            q_b = q_all[b]
            ctx_parts = []
            for hh in range(HQ):
                q_h = q_b[:, hh * DH:(hh + 1) * DH]
                s_parts = []
                for c in range(N_DEV):
                    k_c = kbuf[c, b][:, hh * DH:(hh + 1) * DH]
                    s_parts.append(lax.dot_general(
                        q_h, k_c, (((1,), (1,)), ((), ())),
                        preferred_element_type=jnp.float32))
                scores = jnp.concatenate(s_parts, axis=1) * 0.125
                scores = jnp.where(mask, scores, neg)
                m = jnp.max(scores, axis=1, keepdims=True)
                w = jnp.exp(scores - m)
                w = w / jnp.sum(w, axis=1, keepdims=True)
                ctx_h = jnp.zeros((SQ_PER, DH), jnp.float32)
                for c in range(N_DEV):
                    v_c = vbuf[c, b][:, hh * DH:(hh + 1) * DH]
                    ctx_h = ctx_h + jnp.dot(
                        w[:, c * SKV_PER:(c + 1) * SKV_PER], v_c,
                        preferred_element_type=jnp.float32)
                ctx_parts.append(ctx_h)
            ctx = jnp.concatenate(ctx_parts, axis=1)
            out_ref[b] = jnp.dot(ctx, wo_ref[...],
                                 preferred_element_type=jnp.float32)

    return pl.pallas_call(
        body,
        out_shape=jax.ShapeDtypeStruct((B, SQ_PER, 512), jnp.float32),
        in_specs=[pl.BlockSpec(memory_space=pltpu.VMEM)] * 5,
        out_specs=pl.BlockSpec(memory_space=pltpu.VMEM),
        scratch_shapes=[
            pltpu.VMEM((N_DEV, B, SKV_PER, HD), jnp.float32),
            pltpu.VMEM((N_DEV, B, SKV_PER, HD), jnp.float32),
            pltpu.SemaphoreType.DMA((N_DEV - 1, 2)),
            pltpu.SemaphoreType.DMA((N_DEV - 1, 2)),
            pltpu.SemaphoreType.DMA((N_DEV - 1, 2)),
            pltpu.SemaphoreType.DMA((N_DEV - 1, 2)),
        ],
        compiler_params=pltpu.CompilerParams(collective_id=0),
    )(x, Wq, K_ext, V_ext, Wo)


# device time: 23058 ns/iter; 1.4372x vs baseline; 1.0740x over previous
import jax
import jax.numpy as jnp
from jax import lax
from jax.experimental import pallas as pl
from jax.experimental.pallas import tpu as pltpu

N_DEV = 4
B = 2
SQ_PER = 128
SKV_PER = 128
HQ = 4
DH = 64
HD = HQ * DH
BLK = 64


def kernel(x, Wq, K_ext, V_ext, Wo):
    def body(x_ref, wq_ref, k_ref, v_ref, wo_ref, out_ref,
             kbuf, vbuf, sr, rr, sl, rl):
        my = lax.axis_index("i")
        left = lax.rem(my + N_DEV - 1, N_DEV)
        right = lax.rem(my + 1, N_DEV)

        barrier_sem = pltpu.get_barrier_semaphore()
        for nbr in (left, right):
            pl.semaphore_signal(
                barrier_sem, inc=1,
                device_id=(nbr,), device_id_type=pl.DeviceIdType.MESH,
            )
        pl.semaphore_wait(barrier_sem, 2)

        kbuf[0] = k_ref[...].reshape(B, SKV_PER, HD)
        vbuf[0] = v_ref[...].reshape(B, SKV_PER, HD)

        def start_hop(h):
            rdmas = []
            for buf, sem_i in ((kbuf, 0), (vbuf, 1)):
                rdmas.append(pltpu.make_async_remote_copy(
                    src_ref=buf.at[h, 0], dst_ref=buf.at[h + 1, 0],
                    send_sem=sr.at[h, sem_i], recv_sem=rr.at[h, sem_i],
                    device_id=(right,), device_id_type=pl.DeviceIdType.MESH,
                ))
                rdmas.append(pltpu.make_async_remote_copy(
                    src_ref=buf.at[h, 1], dst_ref=buf.at[h + 1, 1],
                    send_sem=sl.at[h, sem_i], recv_sem=rl.at[h, sem_i],
                    device_id=(left,), device_id_type=pl.DeviceIdType.MESH,
                ))
            for r in rdmas:
                r.start()
            return rdmas

        rdmas = start_hop(0)

        q_all = [jnp.dot(x_ref[b], wq_ref[...],
                         preferred_element_type=jnp.float32)
                 for b in range(B)]

        row = lax.broadcasted_iota(jnp.int32, (SQ_PER, SKV_PER), 0)
        col = lax.broadcasted_iota(jnp.int32, (SQ_PER, SKV_PER), 1)
        qblk = row // BLK + 2 * my
        colblk = col // BLK
        neg = jnp.float32(-1e9)

        state = {}
        for b in range(B):
            for hh in range(HQ):
                state[(b, hh)] = (
                    jnp.full((SQ_PER, 1), neg, jnp.float32),
                    jnp.zeros((SQ_PER, 1), jnp.float32),
                    jnp.zeros((SQ_PER, DH), jnp.float32),
                )

        def proc_slot(b, slot, origin):
            kblk = colblk + 2 * origin
            maskc = (qblk == kblk) | (kblk == 0) | (
                lax.rem(qblk + kblk, 3) == 0)
            k_c = kbuf[slot, b]
            v_c = vbuf[slot, b]
            for hh in range(HQ):
                q_h = q_all[b][:, hh * DH:(hh + 1) * DH]
                s = lax.dot_general(
                    q_h, k_c[:, hh * DH:(hh + 1) * DH],
                    (((1,), (1,)), ((), ())),
                    preferred_element_type=jnp.float32) * 0.125
                s = jnp.where(maskc, s, neg)
                m0, l0, a0 = state[(b, hh)]
                m1 = jnp.maximum(m0, jnp.max(s, axis=1, keepdims=True))
                alpha = jnp.exp(m0 - m1)
                p = jnp.exp(s - m1)
                l1 = l0 * alpha + jnp.sum(p, axis=1, keepdims=True)
                a1 = a0 * alpha + jnp.dot(
                    p, v_c[:, hh * DH:(hh + 1) * DH],
                    preferred_element_type=jnp.float32)
                state[(b, hh)] = (m1, l1, a1)

        proc_slot(0, 0, my)
        proc_slot(1, 0, my)

        for h in range(N_DEV - 1):
            for r in rdmas:
                r.wait()
            if h + 1 < N_DEV - 1:
                rdmas = start_hop(h + 1)
            proc_slot(0, h + 1, lax.rem(my - h - 1 + N_DEV, N_DEV))
            proc_slot(1, h + 1, lax.rem(my + h + 1, N_DEV))

        for b in range(B):
            ctx = jnp.concatenate(
                [state[(b, hh)][2] / state[(b, hh)][1] for hh in range(HQ)],
                axis=1)
            out_ref[b] = jnp.dot(ctx, wo_ref[...],
                                 preferred_element_type=jnp.float32)

    return pl.pallas_call(
        body,
        out_shape=jax.ShapeDtypeStruct((B, SQ_PER, 512), jnp.float32),
        in_specs=[pl.BlockSpec(memory_space=pltpu.VMEM)] * 5,
        out_specs=pl.BlockSpec(memory_space=pltpu.VMEM),
        scratch_shapes=[
            pltpu.VMEM((N_DEV, B, SKV_PER, HD), jnp.float32),
            pltpu.VMEM((N_DEV, B, SKV_PER, HD), jnp.float32),
            pltpu.SemaphoreType.DMA((N_DEV - 1, 2)),
            pltpu.SemaphoreType.DMA((N_DEV - 1, 2)),
            pltpu.SemaphoreType.DMA((N_DEV - 1, 2)),
            pltpu.SemaphoreType.DMA((N_DEV - 1, 2)),
        ],
        compiler_params=pltpu.CompilerParams(collective_id=0),
    )(x, Wq, K_ext, V_ext, Wo)


# device time: 21104 ns/iter; 1.5703x vs baseline; 1.0926x over previous
import jax
import jax.numpy as jnp
from jax import lax
from jax.experimental import pallas as pl
from jax.experimental.pallas import tpu as pltpu

N_DEV = 4
B = 2
SQ_PER = 128
SKV_PER = 128
HQ = 4
DH = 64
HD = HQ * DH
BLK = 64
HALF = SKV_PER // 2


def kernel(x, Wq, K_ext, V_ext, Wo):
    def body(x_ref, wq_ref, k_ref, v_ref, wo_ref, out_ref,
             kbuf, vbuf, s0, r0, s1, r1):
        my = lax.axis_index("i")
        left = lax.rem(my + N_DEV - 1, N_DEV)
        right = lax.rem(my + 1, N_DEV)

        barrier_sem = pltpu.get_barrier_semaphore()
        for nbr in (left, right):
            pl.semaphore_signal(
                barrier_sem, inc=1,
                device_id=(nbr,), device_id_type=pl.DeviceIdType.MESH,
            )
        pl.semaphore_wait(barrier_sem, 2)

        kbuf[0] = k_ref[...].reshape(B, 2, HALF, HD)
        vbuf[0] = v_ref[...].reshape(B, 2, HALF, HD)

        stage0 = []
        for ti, buf in ((0, kbuf), (1, vbuf)):
            stage0.append(pltpu.make_async_remote_copy(
                src_ref=buf.at[0], dst_ref=buf.at[1],
                send_sem=s0.at[0, ti], recv_sem=r0.at[0, ti],
                device_id=(right,), device_id_type=pl.DeviceIdType.MESH,
            ))
            stage0.append(pltpu.make_async_remote_copy(
                src_ref=buf.at[0], dst_ref=buf.at[2],
                send_sem=s0.at[1, ti], recv_sem=r0.at[1, ti],
                device_id=(left,), device_id_type=pl.DeviceIdType.MESH,
            ))
        for r in stage0:
            r.start()

        q_all = [jnp.dot(x_ref[b], wq_ref[...],
                         preferred_element_type=jnp.float32)
                 for b in range(B)]

        row = lax.broadcasted_iota(jnp.int32, (SQ_PER, SKV_PER), 0)
        col = lax.broadcasted_iota(jnp.int32, (SQ_PER, SKV_PER), 1)
        qblk = row // BLK + 2 * my
        colblk = col // BLK
        neg = jnp.float32(-1e9)

        state = {}
        for b in range(B):
            for hh in range(HQ):
                state[(b, hh)] = (
                    jnp.full((SQ_PER, 1), neg, jnp.float32),
                    jnp.zeros((SQ_PER, 1), jnp.float32),
                    jnp.zeros((SQ_PER, DH), jnp.float32),
                )

        def proc_slot(b, slot, origin):
            kblk = colblk + 2 * origin
            maskc = (qblk == kblk) | (kblk == 0) | (
                lax.rem(qblk + kblk, 3) == 0)
            k_c = kbuf[slot, b].reshape(SKV_PER, HD)
            v_c = vbuf[slot, b].reshape(SKV_PER, HD)
            for hh in range(HQ):
                q_h = q_all[b][:, hh * DH:(hh + 1) * DH]
                s = lax.dot_general(
                    q_h, k_c[:, hh * DH:(hh + 1) * DH],
                    (((1,), (1,)), ((), ())),
                    preferred_element_type=jnp.float32) * 0.125
                s = jnp.where(maskc, s, neg)
                m0, l0, a0 = state[(b, hh)]
                m1 = jnp.maximum(m0, jnp.max(s, axis=1, keepdims=True))
                alpha = jnp.exp(m0 - m1)
                p = jnp.exp(s - m1)
                l1 = l0 * alpha + jnp.sum(p, axis=1, keepdims=True)
                a1 = a0 * alpha + jnp.dot(
                    p, v_c[:, hh * DH:(hh + 1) * DH],
                    preferred_element_type=jnp.float32)
                state[(b, hh)] = (m1, l1, a1)

        proc_slot(0, 0, my)
        proc_slot(1, 0, my)

        for r in stage0:
            r.wait()

        stage1 = []
        for ti, buf in ((0, kbuf), (1, vbuf)):
            for b in range(B):
                stage1.append(pltpu.make_async_remote_copy(
                    src_ref=buf.at[1, b, 0], dst_ref=buf.at[3, b, 0],
                    send_sem=s1.at[0, ti, b], recv_sem=r1.at[0, ti, b],
                    device_id=(right,), device_id_type=pl.DeviceIdType.MESH,
                ))
                stage1.append(pltpu.make_async_remote_copy(
                    src_ref=buf.at[2, b, 1], dst_ref=buf.at[3, b, 1],
                    send_sem=s1.at[1, ti, b], recv_sem=r1.at[1, ti, b],
                    device_id=(left,), device_id_type=pl.DeviceIdType.MESH,
                ))
        for r in stage1:
            r.start()

        proc_slot(0, 1, left)
        proc_slot(1, 1, left)
        proc_slot(0, 2, right)
        proc_slot(1, 2, right)

        for r in stage1:
            r.wait()

        diag = lax.rem(my + 2, N_DEV)
        proc_slot(0, 3, diag)
        proc_slot(1, 3, diag)

        for b in range(B):
            ctx = jnp.concatenate(
                [state[(b, hh)][2] / state[(b, hh)][1] for hh in range(HQ)],
                axis=1)
            out_ref[b] = jnp.dot(ctx, wo_ref[...],
                                 preferred_element_type=jnp.float32)

    return pl.pallas_call(
        body,
        out_shape=jax.ShapeDtypeStruct((B, SQ_PER, 512), jnp.float32),
        in_specs=[pl.BlockSpec(memory_space=pltpu.VMEM)] * 5,
        out_specs=pl.BlockSpec(memory_space=pltpu.VMEM),
        scratch_shapes=[
            pltpu.VMEM((N_DEV, B, 2, HALF, HD), jnp.float32),
            pltpu.VMEM((N_DEV, B, 2, HALF, HD), jnp.float32),
            pltpu.SemaphoreType.DMA((2, 2)),
            pltpu.SemaphoreType.DMA((2, 2)),
            pltpu.SemaphoreType.DMA((2, 2, B)),
            pltpu.SemaphoreType.DMA((2, 2, B)),
        ],
        compiler_params=pltpu.CompilerParams(collective_id=0),
    )(x, Wq, K_ext, V_ext, Wo)


# device time: 19482 ns/iter; 1.7011x vs baseline; 1.0833x over previous
import jax
import jax.numpy as jnp
from jax import lax
from jax.experimental import pallas as pl
from jax.experimental.pallas import tpu as pltpu

N_DEV = 4
B = 2
SQ_PER = 128
SKV_PER = 128
HQ = 4
DH = 64
HD = HQ * DH
BLK = 64
HALF = SKV_PER // 2


def kernel(x, Wq, K_ext, V_ext, Wo):
    def body(x_ref, wq_ref, k_ref, v_ref, wo_ref, out_ref,
             kbuf, vbuf, s0, r0, s1, r1):
        my = lax.axis_index("i")
        left = lax.rem(my + N_DEV - 1, N_DEV)
        right = lax.rem(my + 1, N_DEV)

        barrier_sem = pltpu.get_barrier_semaphore()
        for nbr in (left, right):
            pl.semaphore_signal(
                barrier_sem, inc=1,
                device_id=(nbr,), device_id_type=pl.DeviceIdType.MESH,
            )
        pl.semaphore_wait(barrier_sem, 2)

        kbuf[0] = k_ref[...].reshape(B, 2, HALF, HD)
        vbuf[0] = v_ref[...].reshape(B, 2, HALF, HD)

        stage0 = []
        for ti, buf in ((0, kbuf), (1, vbuf)):
            stage0.append(pltpu.make_async_remote_copy(
                src_ref=buf.at[0], dst_ref=buf.at[1],
                send_sem=s0.at[0, ti], recv_sem=r0.at[0, ti],
                device_id=(right,), device_id_type=pl.DeviceIdType.MESH,
            ))
            stage0.append(pltpu.make_async_remote_copy(
                src_ref=buf.at[0], dst_ref=buf.at[2],
                send_sem=s0.at[1, ti], recv_sem=r0.at[1, ti],
                device_id=(left,), device_id_type=pl.DeviceIdType.MESH,
            ))
        for r in stage0:
            r.start()

        q_all = [jnp.dot(x_ref[b], wq_ref[...],
                         preferred_element_type=jnp.float32)
                 for b in range(B)]

        row = lax.broadcasted_iota(jnp.int32, (SQ_PER, SKV_PER), 0)
        col = lax.broadcasted_iota(jnp.int32, (SQ_PER, SKV_PER), 1)
        qblk = row // BLK + 2 * my
        colblk = col // BLK
        neg = jnp.float32(-1e9)

        state = {}
        for b in range(B):
            for hh in range(HQ):
                state[(b, hh)] = (
                    jnp.zeros((SQ_PER, 1), jnp.float32),
                    jnp.zeros((SQ_PER, DH), jnp.float32),
                )

        def proc_slot(b, slot, origin):
            kblk = colblk + 2 * origin
            maskc = ((qblk == kblk) | (kblk == 0) | (
                lax.rem(qblk + kblk, 3) == 0)).astype(jnp.float32)
            k_c = kbuf[slot, b].reshape(SKV_PER, HD)
            v_c = vbuf[slot, b].reshape(SKV_PER, HD)
            for hh in range(HQ):
                q_h = q_all[b][:, hh * DH:(hh + 1) * DH]
                s = lax.dot_general(
                    q_h, k_c[:, hh * DH:(hh + 1) * DH],
                    (((1,), (1,)), ((), ())),
                    preferred_element_type=jnp.float32) * 0.125
                p = jnp.exp(s) * maskc
                l0, a0 = state[(b, hh)]
                l1 = l0 + jnp.sum(p, axis=1, keepdims=True)
                a1 = a0 + jnp.dot(
                    p, v_c[:, hh * DH:(hh + 1) * DH],
                    preferred_element_type=jnp.float32)
                state[(b, hh)] = (l1, a1)

        proc_slot(0, 0, my)
        proc_slot(1, 0, my)

        for r in stage0:
            r.wait()

        stage1 = []
        for ti, buf in ((0, kbuf), (1, vbuf)):
            for b in range(B):
                stage1.append(pltpu.make_async_remote_copy(
                    src_ref=buf.at[1, b, 0], dst_ref=buf.at[3, b, 0],
                    send_sem=s1.at[0, ti, b], recv_sem=r1.at[0, ti, b],
                    device_id=(right,), device_id_type=pl.DeviceIdType.MESH,
                ))
                stage1.append(pltpu.make_async_remote_copy(
                    src_ref=buf.at[2, b, 1], dst_ref=buf.at[3, b, 1],
                    send_sem=s1.at[1, ti, b], recv_sem=r1.at[1, ti, b],
                    device_id=(left,), device_id_type=pl.DeviceIdType.MESH,
                ))
        for r in stage1:
            r.start()

        proc_slot(0, 1, left)
        proc_slot(1, 1, left)
        proc_slot(0, 2, right)
        proc_slot(1, 2, right)

        for r in stage1:
            r.wait()

        diag = lax.rem(my + 2, N_DEV)
        proc_slot(0, 3, diag)
        proc_slot(1, 3, diag)

        for b in range(B):
            ctx = jnp.concatenate(
                [state[(b, hh)][1] / state[(b, hh)][0] for hh in range(HQ)],
                axis=1)
            out_ref[b] = jnp.dot(ctx, wo_ref[...],
                                 preferred_element_type=jnp.float32)

    return pl.pallas_call(
        body,
        out_shape=jax.ShapeDtypeStruct((B, SQ_PER, 512), jnp.float32),
        in_specs=[pl.BlockSpec(memory_space=pltpu.VMEM)] * 5,
        out_specs=pl.BlockSpec(memory_space=pltpu.VMEM),
        scratch_shapes=[
            pltpu.VMEM((N_DEV, B, 2, HALF, HD), jnp.float32),
            pltpu.VMEM((N_DEV, B, 2, HALF, HD), jnp.float32),
            pltpu.SemaphoreType.DMA((2, 2)),
            pltpu.SemaphoreType.DMA((2, 2)),
            pltpu.SemaphoreType.DMA((2, 2, B)),
            pltpu.SemaphoreType.DMA((2, 2, B)),
        ],
        compiler_params=pltpu.CompilerParams(collective_id=0),
    )(x, Wq, K_ext, V_ext, Wo)


# device time: 18292 ns/iter; 1.8117x vs baseline; 1.0651x over previous
import jax
import jax.numpy as jnp
from jax import lax
from jax.experimental import pallas as pl
from jax.experimental.pallas import tpu as pltpu

N_DEV = 4
B = 2
SQ_PER = 128
SKV_PER = 128
HQ = 4
DH = 64
HD = HQ * DH
BLK = 64
HALF = SKV_PER // 2


def kernel(x, Wq, K_ext, V_ext, Wo):
    def body(x_ref, wq_ref, k_ref, v_ref, wo_ref, out_ref,
             kbuf, vbuf, s0, r0, s1, r1):
        my = lax.axis_index("i")
        left = lax.rem(my + N_DEV - 1, N_DEV)
        right = lax.rem(my + 1, N_DEV)

        barrier_sem = pltpu.get_barrier_semaphore()
        for nbr in (left, right):
            pl.semaphore_signal(
                barrier_sem, inc=1,
                device_id=(nbr,), device_id_type=pl.DeviceIdType.MESH,
            )
        pl.semaphore_wait(barrier_sem, 2)

        kbuf[0] = k_ref[...].reshape(B, 2, HALF, HD)
        vbuf[0] = v_ref[...].reshape(B, 2, HALF, HD)

        def start_wave0(w):
            rs = []
            for ti, buf in ((0, kbuf), (1, vbuf)):
                rs.append(pltpu.make_async_remote_copy(
                    src_ref=buf.at[0, w], dst_ref=buf.at[1, w],
                    send_sem=s0.at[w, 0, ti], recv_sem=r0.at[w, 0, ti],
                    device_id=(right,), device_id_type=pl.DeviceIdType.MESH,
                ))
                rs.append(pltpu.make_async_remote_copy(
                    src_ref=buf.at[0, 1 - w], dst_ref=buf.at[2, 1 - w],
                    send_sem=s0.at[w, 1, ti], recv_sem=r0.at[w, 1, ti],
                    device_id=(left,), device_id_type=pl.DeviceIdType.MESH,
                ))
            for r in rs:
                r.start()
            return rs

        def start_wave1(w):
            rs = []
            for ti, buf in ((0, kbuf), (1, vbuf)):
                rs.append(pltpu.make_async_remote_copy(
                    src_ref=buf.at[1, w, 0], dst_ref=buf.at[3, w, 0],
                    send_sem=s1.at[w, 0, ti], recv_sem=r1.at[w, 0, ti],
                    device_id=(right,), device_id_type=pl.DeviceIdType.MESH,
                ))
                rs.append(pltpu.make_async_remote_copy(
                    src_ref=buf.at[2, 1 - w, 1], dst_ref=buf.at[3, 1 - w, 1],
                    send_sem=s1.at[w, 1, ti], recv_sem=r1.at[w, 1, ti],
                    device_id=(left,), device_id_type=pl.DeviceIdType.MESH,
                ))
            for r in rs:
                r.start()
            return rs

        wave_a = start_wave0(0)
        wave_b = start_wave0(1)

        q_all = [jnp.dot(x_ref[b], wq_ref[...],
                         preferred_element_type=jnp.float32)
                 for b in range(B)]

        row = lax.broadcasted_iota(jnp.int32, (SQ_PER, SKV_PER), 0)
        col = lax.broadcasted_iota(jnp.int32, (SQ_PER, SKV_PER), 1)
        qblk = row // BLK + 2 * my
        colblk = col // BLK

        state = {}
        for b in range(B):
            for hh in range(HQ):
                state[(b, hh)] = (
                    jnp.zeros((SQ_PER, 1), jnp.float32),
                    jnp.zeros((SQ_PER, DH), jnp.float32),
                )

        def proc_slot(b, slot, origin):
            kblk = colblk + 2 * origin
            maskc = ((qblk == kblk) | (kblk == 0) | (
                lax.rem(qblk + kblk, 3) == 0)).astype(jnp.float32)
            k_c = kbuf[slot, b].reshape(SKV_PER, HD)
            v_c = vbuf[slot, b].reshape(SKV_PER, HD)
            for hh in range(HQ):
                q_h = q_all[b][:, hh * DH:(hh + 1) * DH]
                s = lax.dot_general(
                    q_h, k_c[:, hh * DH:(hh + 1) * DH],
                    (((1,), (1,)), ((), ())),
                    preferred_element_type=jnp.float32) * 0.125
                p = jnp.exp(s) * maskc
                l0, a0 = state[(b, hh)]
                l1 = l0 + jnp.sum(p, axis=1, keepdims=True)
                a1 = a0 + jnp.dot(
                    p, v_c[:, hh * DH:(hh + 1) * DH],
                    preferred_element_type=jnp.float32)
                state[(b, hh)] = (l1, a1)

        proc_slot(0, 0, my)
        proc_slot(1, 0, my)

        for r in wave_a:
            r.wait()
        wave_1a = start_wave1(0)
        proc_slot(0, 1, left)
        proc_slot(1, 2, right)

        for r in wave_b:
            r.wait()
        wave_1b = start_wave1(1)
        proc_slot(1, 1, left)
        proc_slot(0, 2, right)

        for r in wave_1a + wave_1b:
            r.wait()
        diag = lax.rem(my + 2, N_DEV)
        proc_slot(0, 3, diag)
        proc_slot(1, 3, diag)

        for b in range(B):
            ctx = jnp.concatenate(
                [state[(b, hh)][1] / state[(b, hh)][0] for hh in range(HQ)],
                axis=1)
            out_ref[b] = jnp.dot(ctx, wo_ref[...],
                                 preferred_element_type=jnp.float32)

    return pl.pallas_call(
        body,
        out_shape=jax.ShapeDtypeStruct((B, SQ_PER, 512), jnp.float32),
        in_specs=[pl.BlockSpec(memory_space=pltpu.VMEM)] * 5,
        out_specs=pl.BlockSpec(memory_space=pltpu.VMEM),
        scratch_shapes=[
            pltpu.VMEM((N_DEV, B, 2, HALF, HD), jnp.float32),
            pltpu.VMEM((N_DEV, B, 2, HALF, HD), jnp.float32),
            pltpu.SemaphoreType.DMA((2, 2, 2)),
            pltpu.SemaphoreType.DMA((2, 2, 2)),
            pltpu.SemaphoreType.DMA((2, 2, 2)),
            pltpu.SemaphoreType.DMA((2, 2, 2)),
        ],
        compiler_params=pltpu.CompilerParams(collective_id=0),
    )(x, Wq, K_ext, V_ext, Wo)


# device time: 18045 ns/iter; 1.8365x vs baseline; 1.0137x over previous
import jax
import jax.numpy as jnp
from jax import lax
from jax.experimental import pallas as pl
from jax.experimental.pallas import tpu as pltpu

N_DEV = 4
B = 2
SQ_PER = 128
SKV_PER = 128
HQ = 4
DH = 64
HD = HQ * DH
BLK = 64
HALF = SKV_PER // 2


def kernel(x, Wq, K_ext, V_ext, Wo):
    def body(x_ref, wq_ref, k_ref, v_ref, wo_ref, out_ref,
             kbuf, vbuf, s0, r0, s1, r1):
        my = lax.axis_index("i")
        left = lax.rem(my + N_DEV - 1, N_DEV)
        right = lax.rem(my + 1, N_DEV)

        barrier_sem = pltpu.get_barrier_semaphore()
        for nbr in (left, right):
            pl.semaphore_signal(
                barrier_sem, inc=1,
                device_id=(nbr,), device_id_type=pl.DeviceIdType.MESH,
            )
        pl.semaphore_wait(barrier_sem, 2)

        kbuf[0] = k_ref[...].reshape(B, 2, HALF, HD)
        vbuf[0] = v_ref[...].reshape(B, 2, HALF, HD)

        def start_wave0(w):
            rs = []
            for ti, buf in ((0, kbuf), (1, vbuf)):
                rs.append(pltpu.make_async_remote_copy(
                    src_ref=buf.at[0, w], dst_ref=buf.at[1, w],
                    send_sem=s0.at[w, 0, ti], recv_sem=r0.at[w, 0, ti],
                    device_id=(right,), device_id_type=pl.DeviceIdType.MESH,
                ))
                rs.append(pltpu.make_async_remote_copy(
                    src_ref=buf.at[0, 1 - w], dst_ref=buf.at[2, 1 - w],
                    send_sem=s0.at[w, 1, ti], recv_sem=r0.at[w, 1, ti],
                    device_id=(left,), device_id_type=pl.DeviceIdType.MESH,
                ))
            for r in rs:
                r.start()
            return rs

        def start_wave1(w):
            rs = []
            for ti, buf in ((0, kbuf), (1, vbuf)):
                rs.append(pltpu.make_async_remote_copy(
                    src_ref=buf.at[1, w, 0], dst_ref=buf.at[3, w, 0],
                    send_sem=s1.at[w, 0, ti], recv_sem=r1.at[w, 0, ti],
                    device_id=(right,), device_id_type=pl.DeviceIdType.MESH,
                ))
                rs.append(pltpu.make_async_remote_copy(
                    src_ref=buf.at[2, 1 - w, 1], dst_ref=buf.at[3, 1 - w, 1],
                    send_sem=s1.at[w, 1, ti], recv_sem=r1.at[w, 1, ti],
                    device_id=(left,), device_id_type=pl.DeviceIdType.MESH,
                ))
            for r in rs:
                r.start()
            return rs

        wave_a = start_wave0(0)
        wave_b = start_wave0(1)

        q_all = [jnp.dot(x_ref[b], wq_ref[...],
                         preferred_element_type=jnp.float32)
                 for b in range(B)]

        row = lax.broadcasted_iota(jnp.int32, (SQ_PER, SKV_PER), 0)
        col = lax.broadcasted_iota(jnp.int32, (SQ_PER, SKV_PER), 1)
        qblk = row // BLK + 2 * my
        colblk = col // BLK

        state = {}
        for b in range(B):
            for hh in range(HQ):
                state[(b, hh)] = (
                    jnp.zeros((SQ_PER, 1), jnp.float32),
                    jnp.zeros((SQ_PER, DH), jnp.float32),
                )

        def chunk_mask(origin):
            kblk = colblk + 2 * origin
            return ((qblk == kblk) | (kblk == 0) | (
                lax.rem(qblk + kblk, 3) == 0)).astype(jnp.float32)

        def proc_slot(b, slot, maskc):
            k_c = kbuf[slot, b].reshape(SKV_PER, HD)
            v_c = vbuf[slot, b].reshape(SKV_PER, HD)
            for hh in range(HQ):
                q_h = q_all[b][:, hh * DH:(hh + 1) * DH]
                s = lax.dot_general(
                    q_h, k_c[:, hh * DH:(hh + 1) * DH],
                    (((1,), (1,)), ((), ())),
                    preferred_element_type=jnp.float32) * 0.125
                p = jnp.exp(s) * maskc
                l0, a0 = state[(b, hh)]
                l1 = l0 + jnp.sum(p, axis=1, keepdims=True)
                a1 = a0 + jnp.dot(
                    p, v_c[:, hh * DH:(hh + 1) * DH],
                    preferred_element_type=jnp.float32)
                state[(b, hh)] = (l1, a1)

        def proc_half(b, half, origin):
            kb = 2 * origin + half
            qrow = qblk[:, :1]
            maskh = ((qrow == kb) | (kb == 0) | (
                lax.rem(qrow + kb, 3) == 0)).astype(jnp.float32)
            k_c = kbuf[3, b, half]
            v_c = vbuf[3, b, half]
            for hh in range(HQ):
                q_h = q_all[b][:, hh * DH:(hh + 1) * DH]
                s = lax.dot_general(
                    q_h, k_c[:, hh * DH:(hh + 1) * DH],
                    (((1,), (1,)), ((), ())),
                    preferred_element_type=jnp.float32) * 0.125
                p = jnp.exp(s) * maskh
                l0, a0 = state[(b, hh)]
                l1 = l0 + jnp.sum(p, axis=1, keepdims=True)
                a1 = a0 + jnp.dot(
                    p, v_c[:, hh * DH:(hh + 1) * DH],
                    preferred_element_type=jnp.float32)
                state[(b, hh)] = (l1, a1)

        mask_own = chunk_mask(my)
        proc_slot(0, 0, mask_own)
        proc_slot(1, 0, mask_own)

        for r in wave_a:
            r.wait()
        wave_1a = start_wave1(0)
        mask_left = chunk_mask(left)
        mask_right = chunk_mask(right)
        proc_slot(0, 1, mask_left)
        proc_slot(1, 2, mask_right)

        for r in wave_b:
            r.wait()
        wave_1b = start_wave1(1)
        proc_slot(1, 1, mask_left)
        proc_slot(0, 2, mask_right)

        diag = lax.rem(my + 2, N_DEV)
        for r in wave_1a:
            r.wait()
        proc_half(0, 0, diag)
        proc_half(1, 1, diag)
        for r in wave_1b:
            r.wait()
        proc_half(1, 0, diag)
        proc_half(0, 1, diag)

        for b in range(B):
            ctx = jnp.concatenate(
                [state[(b, hh)][1] / state[(b, hh)][0] for hh in range(HQ)],
                axis=1)
            out_ref[b] = jnp.dot(ctx, wo_ref[...],
                                 preferred_element_type=jnp.float32)

    return pl.pallas_call(
        body,
        out_shape=jax.ShapeDtypeStruct((B, SQ_PER, 512), jnp.float32),
        in_specs=[pl.BlockSpec(memory_space=pltpu.VMEM)] * 5,
        out_specs=pl.BlockSpec(memory_space=pltpu.VMEM),
        scratch_shapes=[
            pltpu.VMEM((N_DEV, B, 2, HALF, HD), jnp.float32),
            pltpu.VMEM((N_DEV, B, 2, HALF, HD), jnp.float32),
            pltpu.SemaphoreType.DMA((2, 2, 2)),
            pltpu.SemaphoreType.DMA((2, 2, 2)),
            pltpu.SemaphoreType.DMA((2, 2, 2)),
            pltpu.SemaphoreType.DMA((2, 2, 2)),
        ],
        compiler_params=pltpu.CompilerParams(collective_id=0),
    )(x, Wq, K_ext, V_ext, Wo)
